# baseline (speedup 1.0000x reference)
import functools

import numpy as np
import jax
import jax.numpy as jnp
from jax import lax
from jax.experimental import pallas as pl
from jax.experimental.pallas import tpu as pltpu
from jax.experimental.pallas import tpu_sc as plsc

_NN = 10000
_NE = 160000
_C = 128
_MSG = 288
_CW = _MSG // 4
_CH = 128
_NCH = _NE // _CH


def _build_expand():
    exp = [np.zeros((32, _MSG), np.float32) for _ in range(3)]
    fexp = np.zeros((9, _MSG), np.float32)
    for j in range(_MSG):
        if j < 32:
            l, m, k = 0, j, 0
        elif j < 128:
            l, m, k = 1, (j - 32) // 3, (j - 32) % 3
        else:
            l, m, k = 2, (j - 128) // 5, (j - 128) % 5
        exp[l][m, j] = 1.0
        fexp[(0, 1, 4)[l] + k, j] = 1.0
    return exp[0], exp[1], exp[2], fexp


_EXP0, _EXP1, _EXP2, _FEXP = _build_expand()


def _build_sh_affine():
    c1 = np.sqrt(3.0)
    c2 = np.sqrt(15.0)
    c2b = np.sqrt(5.0) / 2.0
    A = np.zeros((3, 9), np.float32)
    B = np.zeros((3, 9), np.float32)
    ar = np.zeros((9,), np.float32)
    br = np.zeros((9,), np.float32)
    cr = np.zeros((9,), np.float32)
    ar[0] = 1.0
    br[0] = 1.0
    for j, ax in ((1, 0), (2, 1), (3, 2)):
        A[ax, j] = 1.0
        br[j] = c1
    A[0, 4] = 1.0; B[1, 4] = c2
    A[1, 5] = 1.0; B[2, 5] = c2
    A[2, 6] = 1.0; B[2, 6] = 3.0 * c2b
    cr[6] = -c2b
    A[0, 7] = 1.0; B[2, 7] = c2
    A[0, 8] = 1.0; A[1, 8] = -1.0
    B[0, 8] = c2 / 2.0; B[1, 8] = c2 / 2.0
    return A, B, ar.reshape(1, 9), br.reshape(1, 9), cr.reshape(1, 9)


_SH_A, _SH_B, _SH_AR, _SH_BR, _SH_CR = _build_sh_affine()

_SC_PARAMS = pltpu.CompilerParams(use_tc_tiling_on_sc=False,
                                  needs_layout_passes=False)
_IOTA16 = tuple(range(16))


def _extract_col(ei_buf, idx_buf, col):
    cols = jnp.full((16,), col, jnp.int32)
    for g in range(_CH // 16):
        rows = lax.iota(jnp.int32, 16) + g * 16
        v = plsc.load_gather(ei_buf, [rows, cols])
        idx_buf[pl.ds(g * 16, 16)] = v


def _sc_gather(table, ei):
    mesh = plsc.VectorSubcoreMesh(core_axis_name="c", subcore_axis_name="s")

    @functools.partial(
        pl.kernel,
        mesh=mesh,
        compiler_params=_SC_PARAMS,
        out_type=jax.ShapeDtypeStruct((_NE, _C), jnp.bfloat16),
        scratch_types=[
            pltpu.VMEM((_CH, 2), jnp.int32),
            pltpu.VMEM((_CH, 2), jnp.int32),
            pltpu.VMEM((_CH,), jnp.int32),
            pltpu.VMEM((_CH,), jnp.int32),
            pltpu.VMEM((_CH, _C), jnp.bfloat16),
            pltpu.VMEM((_CH, _C), jnp.bfloat16),
            pltpu.SemaphoreType.DMA,
            pltpu.SemaphoreType.DMA,
            pltpu.SemaphoreType.DMA,
            pltpu.SemaphoreType.DMA,
            pltpu.SemaphoreType.DMA,
            pltpu.SemaphoreType.DMA,
        ],
    )
    def k(table_hbm, ei_hbm, out_hbm, ei0, ei1, ix0, ix1, rw0, rw1,
          se0, se1, sg0, sg1, ss0, ss1):
        wid = lax.axis_index("s") * 2 + lax.axis_index("c")
        eib = (ei0, ei1)
        ixb = (ix0, ix1)
        rwb = (rw0, rw1)
        se = (se0, se1)
        sg = (sg0, sg1)
        ss = (ss0, ss1)
        nj = 39 + jnp.where(wid < 2, 1, 0)

        def chunk_off(j):
            return (wid + j * 32) * _CH

        pltpu.async_copy(ei_hbm.at[pl.ds(chunk_off(0), _CH)], ei0, se0)

        def iteration(j, b):
            nb = 1 - b
            pltpu.make_async_copy(
                ei_hbm.at[pl.ds(0, _CH)], eib[b], se[b]).wait()
            _extract_col(eib[b], ixb[b], 0)

            @pl.when(j >= 2)
            def _():
                pltpu.make_async_copy(
                    rwb[b], out_hbm.at[pl.ds(0, _CH)], ss[b]).wait()

            pltpu.async_copy(table_hbm.at[ixb[b]], rwb[b], sg[b])

            @pl.when(j + 1 < nj)
            def _():
                pltpu.async_copy(
                    ei_hbm.at[pl.ds(chunk_off(j + 1), _CH)], eib[nb], se[nb])

            pltpu.make_async_copy(table_hbm.at[ixb[b]], rwb[b], sg[b]).wait()
            pltpu.async_copy(rwb[b], out_hbm.at[pl.ds(chunk_off(j), _CH)],
                             ss[b])

        def body(j, carry):
            @pl.when(j % 2 == 0)
            def _():
                iteration(j, 0)

            @pl.when(j % 2 == 1)
            def _():
                iteration(j, 1)
            return carry

        lax.fori_loop(0, nj, body, 0)
        pltpu.make_async_copy(rw0, out_hbm.at[pl.ds(0, _CH)], ss0).wait()
        pltpu.make_async_copy(rw1, out_hbm.at[pl.ds(0, _CH)], ss1).wait()

    return k(table, ei)


def _tc_messages(ef, ev, src, W1, W2, W3, Q, fexp, sha, shb):
    BE = 2000
    grid = _NE // BE

    def body(ef_r, ev_r, src_r, W1_r, W2_r, W3_r, Q_r, f_r, a_r, b_r, out_r):
        bf = jnp.bfloat16
        f32 = jnp.float32
        h = jax.nn.silu(jnp.dot(ef_r[...].astype(bf), W1_r[...],
                                preferred_element_type=f32))
        h = jax.nn.silu(jnp.dot(h.astype(bf), W2_r[...],
                                preferred_element_type=f32))
        w = jax.nn.silu(jnp.dot(h.astype(bf), W3_r[...],
                                preferred_element_type=f32))
        s = src_r[...]
        g = w.astype(bf) * jnp.concatenate([s, s, s], axis=1)
        acc = jnp.dot(g, Q_r[...],
                      preferred_element_type=f32)
        v = ev_r[...]
        n = jnp.sqrt(jnp.sum(v * v, axis=1, keepdims=True))
        vn = v / jnp.maximum(n, 1e-9)
        u = vn @ a_r[0:3] + a_r[3:4]
        t = vn @ b_r[0:3] + b_r[3:4]
        y9 = u * t + a_r[4:5]
        y288 = y9 @ f_r[...]
        out_r[...] = acc * y288

    full = lambda a, b: pl.BlockSpec((a, b), lambda i: (0, 0))
    return pl.pallas_call(
        body,
        grid=(grid,),
        in_specs=[
            pl.BlockSpec((BE, 16), lambda i: (i, 0)),
            pl.BlockSpec((BE, 3), lambda i: (i, 0)),
            pl.BlockSpec((BE, _C), lambda i: (i, 0)),
            full(16, 64), full(64, 64), full(64, 384),
            full(384, _MSG), full(9, _MSG), full(5, 9), full(4, 9),
        ],
        out_specs=pl.BlockSpec((BE, _MSG), lambda i: (i, 0)),
        out_shape=jax.ShapeDtypeStruct((_NE, _MSG), jnp.float32),
    )(ef, ev, src, W1, W2, W3, Q, fexp, sha, shb)


def _sc_scatter(msg, ei, z25):
    mesh = plsc.VectorSubcoreMesh(core_axis_name="c", subcore_axis_name="s")

    @functools.partial(
        pl.kernel,
        mesh=mesh,
        compiler_params=_SC_PARAMS,
        out_type=jax.ShapeDtypeStruct((_NN, _MSG), jnp.float32),
        scratch_types=[
            pltpu.VMEM((_CH, 2), jnp.int32),
            pltpu.VMEM((_CH, 2), jnp.int32),
            pltpu.VMEM((_CH,), jnp.int32),
            pltpu.VMEM((_CH,), jnp.int32),
            pltpu.VMEM((_CH, _CW), jnp.float32),
            pltpu.VMEM((_CH, _CW), jnp.float32),
            pltpu.VMEM((25, _CW), jnp.float32),
            pltpu.VMEM_SHARED((_NN, _CW), jnp.float32),
            pltpu.SemaphoreType.DMA,
            pltpu.SemaphoreType.DMA,
            pltpu.SemaphoreType.DMA,
            pltpu.SemaphoreType.DMA,
            pltpu.SemaphoreType.DMA,
            pltpu.SemaphoreType.DMA,
        ],
    )
    def k(msg_hbm, ei_hbm, z_hbm, out_hbm, ei0, ei1, ix0, ix1, m0, m1,
          zbuf, acc, se0, se1, sl0, sl1, sa0, sa1):
        c = lax.axis_index("c")
        sid = lax.axis_index("s")
        eib = (ei0, ei1)
        ixb = (ix0, ix1)
        mb = (m0, m1)
        se = (se0, se1)
        sl = (sl0, sl1)
        sa = (sa0, sa1)

        nj = 78 + jnp.where(sid < 2, 1, 0)

        def chunk_off(j):
            return (sid + j * 16) * _CH

        pltpu.sync_copy(z_hbm, zbuf)

        for p in range(2):
            colbase = (c + 2 * p) * _CW

            def zbody(kk, carry):
                pltpu.sync_copy(zbuf, acc.at[pl.ds(sid * 625 + kk * 25, 25)])
                return carry

            lax.fori_loop(0, 25, zbody, 0)
            plsc.subcore_barrier()

            pltpu.async_copy(ei_hbm.at[pl.ds(chunk_off(0), _CH)], ei0, se0)
            pltpu.async_copy(
                msg_hbm.at[pl.ds(chunk_off(0), _CH), pl.ds(colbase, _CW)],
                m0, sl0)

            def iteration(j, b):
                nb = 1 - b
                pltpu.make_async_copy(
                    ei_hbm.at[pl.ds(0, _CH)], eib[b], se[b]).wait()
                _extract_col(eib[b], ixb[b], 1)
                pltpu.make_async_copy(
                    msg_hbm.at[pl.ds(0, _CH), pl.ds(0, _CW)], mb[b],
                    sl[b]).wait()
                pltpu.async_copy(mb[b], acc.at[ixb[b]], sa[b], add=True)

                @pl.when(j + 1 < nj)
                def _():
                    @pl.when(j >= 1)
                    def _():
                        pltpu.make_async_copy(mb[nb], acc.at[ixb[nb]],
                                              sa[nb]).wait()
                    off = chunk_off(j + 1)
                    pltpu.async_copy(ei_hbm.at[pl.ds(off, _CH)], eib[nb],
                                     se[nb])
                    pltpu.async_copy(
                        msg_hbm.at[pl.ds(off, _CH), pl.ds(colbase, _CW)],
                        mb[nb], sl[nb])

            def body(j, carry):
                @pl.when(j % 2 == 0)
                def _():
                    iteration(j, 0)

                @pl.when(j % 2 == 1)
                def _():
                    iteration(j, 1)
                return carry

            lax.fori_loop(0, nj, body, 0)
            pltpu.make_async_copy(m0, acc.at[ix0], sa0).wait()
            pltpu.make_async_copy(m1, acc.at[ix1], sa1).wait()
            plsc.subcore_barrier()

            def wbody(kk, carry):
                r0 = sid * 625 + kk * 125
                pltpu.sync_copy(acc.at[pl.ds(r0, 125)], m0.at[pl.ds(0, 125)])
                pltpu.sync_copy(
                    m0.at[pl.ds(0, 125)],
                    out_hbm.at[pl.ds(r0, 125), pl.ds(colbase, _CW)])
                return carry

            lax.fori_loop(0, 5, wbody, 0)
            plsc.subcore_barrier()

    return k(msg, ei, z25)


def kernel(node_features, edge_features, edge_vectors, edge_index, W1, W2, W3, P):
    bf = jnp.bfloat16
    src = _sc_gather(node_features.astype(bf), edge_index)
    Q = jnp.concatenate(
        [P[0].T @ _EXP0, P[1].T @ _EXP1, P[2].T @ _EXP2], axis=0).astype(bf)
    sha = jnp.concatenate(
        [jnp.asarray(_SH_A), jnp.asarray(_SH_AR), jnp.asarray(_SH_CR)], axis=0)
    shb = jnp.concatenate([jnp.asarray(_SH_B), jnp.asarray(_SH_BR)], axis=0)
    msg = _tc_messages(
        edge_features, edge_vectors, src, W1.astype(bf), W2.astype(bf),
        W3.astype(bf), Q, jnp.asarray(_FEXP), sha, shb)
    z25 = jnp.zeros((25, _CW), jnp.float32)
    return _sc_scatter(msg, edge_index, z25)


# speedup vs baseline: 1.5429x; 1.5429x over previous
import functools

import numpy as np
import jax
import jax.numpy as jnp
from jax import lax
from jax.experimental import pallas as pl
from jax.experimental.pallas import tpu as pltpu
from jax.experimental.pallas import tpu_sc as plsc

_NN = 10000
_NE = 160000
_C = 128
_MSG = 288
_CW = 72
_CH = 128
_NCH = _NE // _CH

_PIECES = {
    (0, 0): ((0, 0, 0, 72),),
    (1, 0): ((0, 72, 0, 56), (1, 0, 56, 16)),
    (0, 1): ((1, 16, 0, 72),),
    (1, 1): ((1, 88, 0, 40), (2, 0, 40, 32)),
}


def _build_expand():
    exp = [np.zeros((32, _MSG), np.float32) for _ in range(3)]
    fexp = np.zeros((9, _MSG), np.float32)
    for j in range(_MSG):
        if j < 32:
            l, m, k = 0, j, 0
        elif j < 128:
            l, m, k = 1, (j - 32) // 3, (j - 32) % 3
        else:
            l, m, k = 2, (j - 128) // 5, (j - 128) % 5
        exp[l][m, j] = 1.0
        fexp[(0, 1, 4)[l] + k, j] = 1.0
    return exp[0], exp[1], exp[2], fexp


_EXP0, _EXP1, _EXP2, _FEXP = _build_expand()


def _build_sh_affine():
    c1 = np.sqrt(3.0)
    c2 = np.sqrt(15.0)
    c2b = np.sqrt(5.0) / 2.0
    A = np.zeros((3, 9), np.float32)
    B = np.zeros((3, 9), np.float32)
    ar = np.zeros((9,), np.float32)
    br = np.zeros((9,), np.float32)
    cr = np.zeros((9,), np.float32)
    ar[0] = 1.0
    br[0] = 1.0
    for j, ax in ((1, 0), (2, 1), (3, 2)):
        A[ax, j] = 1.0
        br[j] = c1
    A[0, 4] = 1.0; B[1, 4] = c2
    A[1, 5] = 1.0; B[2, 5] = c2
    A[2, 6] = 1.0; B[2, 6] = 3.0 * c2b
    cr[6] = -c2b
    A[0, 7] = 1.0; B[2, 7] = c2
    A[0, 8] = 1.0; A[1, 8] = -1.0
    B[0, 8] = c2 / 2.0; B[1, 8] = c2 / 2.0
    return A, B, ar.reshape(1, 9), br.reshape(1, 9), cr.reshape(1, 9)


_SH_A, _SH_B, _SH_AR, _SH_BR, _SH_CR = _build_sh_affine()

_SC_PARAMS = pltpu.CompilerParams(use_tc_tiling_on_sc=False,
                                  needs_layout_passes=False)


def _sc_gather(table, idx):
    mesh = plsc.VectorSubcoreMesh(core_axis_name="c", subcore_axis_name="s")

    @functools.partial(
        pl.kernel,
        mesh=mesh,
        compiler_params=_SC_PARAMS,
        out_type=jax.ShapeDtypeStruct((_NE, _C), jnp.float32),
        scratch_types=[
            pltpu.VMEM((_CH,), jnp.int32),
            pltpu.VMEM((_CH,), jnp.int32),
            pltpu.VMEM((_CH, _C), jnp.float32),
            pltpu.VMEM((_CH, _C), jnp.float32),
            pltpu.SemaphoreType.DMA,
            pltpu.SemaphoreType.DMA,
            pltpu.SemaphoreType.DMA,
            pltpu.SemaphoreType.DMA,
            pltpu.SemaphoreType.DMA,
            pltpu.SemaphoreType.DMA,
        ],
    )
    def k(table_hbm, idx_hbm, out_hbm, ix0, ix1, rw0, rw1,
          se0, se1, sg0, sg1, ss0, ss1):
        wid = lax.axis_index("s") * 2 + lax.axis_index("c")
        ixb = (ix0, ix1)
        rwb = (rw0, rw1)
        se = (se0, se1)
        sg = (sg0, sg1)
        ss = (ss0, ss1)
        nj = 39 + jnp.where(wid < 2, 1, 0)

        def chunk_off(j):
            return (wid + j * 32) * _CH

        pltpu.async_copy(idx_hbm.at[pl.ds(chunk_off(0), _CH)], ix0, se0)

        def iteration(j, b):
            nb = 1 - b
            pltpu.make_async_copy(
                idx_hbm.at[pl.ds(0, _CH)], ixb[b], se[b]).wait()

            @pl.when(j >= 2)
            def _():
                pltpu.make_async_copy(
                    rwb[b], out_hbm.at[pl.ds(0, _CH)], ss[b]).wait()

            pltpu.async_copy(table_hbm.at[ixb[b]], rwb[b], sg[b])

            @pl.when(j + 1 < nj)
            def _():
                pltpu.async_copy(
                    idx_hbm.at[pl.ds(chunk_off(j + 1), _CH)], ixb[nb], se[nb])

            pltpu.make_async_copy(table_hbm.at[ixb[b]], rwb[b], sg[b]).wait()
            pltpu.async_copy(rwb[b], out_hbm.at[pl.ds(chunk_off(j), _CH)],
                             ss[b])

        def body(j, carry):
            @pl.when(j % 2 == 0)
            def _():
                iteration(j, 0)

            @pl.when(j % 2 == 1)
            def _():
                iteration(j, 1)
            return carry

        lax.fori_loop(0, nj, body, 0)
        pltpu.make_async_copy(rw0, out_hbm.at[pl.ds(0, _CH)], ss0).wait()
        pltpu.make_async_copy(rw1, out_hbm.at[pl.ds(0, _CH)], ss1).wait()

    return k(table, idx)


def _tc_messages(ef, ev, src, W1, W2, W3, Q, fexp, sha, shb):
    BE = 2000
    grid = _NE // BE

    def body(ef_r, ev_r, src_r, W1_r, W2_r, W3_r, Q_r, f_r, a_r, b_r,
             o0_r, o1_r, o2_r):
        bf = jnp.bfloat16
        f32 = jnp.float32
        h = jax.nn.silu(jnp.dot(ef_r[...].astype(bf), W1_r[...],
                                preferred_element_type=f32))
        h = jax.nn.silu(jnp.dot(h.astype(bf), W2_r[...],
                                preferred_element_type=f32))
        w = jax.nn.silu(jnp.dot(h.astype(bf), W3_r[...],
                                preferred_element_type=f32))
        s = src_r[...]
        g = w * jnp.concatenate([s, s, s], axis=1)
        acc = jnp.dot(g.astype(bf), Q_r[...],
                      preferred_element_type=f32)
        v = ev_r[...]
        n = jnp.sqrt(jnp.sum(v * v, axis=1, keepdims=True))
        vn = v / jnp.maximum(n, 1e-9)
        u = vn @ a_r[0:3] + a_r[3:4]
        t = vn @ b_r[0:3] + b_r[3:4]
        y9 = u * t + a_r[4:5]
        y288 = y9 @ f_r[...]
        res = acc * y288
        o0_r[...] = res[:, 0:128]
        o1_r[...] = res[:, 128:256]
        o2_r[:, 0:32] = res[:, 256:288]

    full = lambda a, b: pl.BlockSpec((a, b), lambda i: (0, 0))
    eb = lambda: pl.BlockSpec((BE, _C), lambda i: (i, 0))
    return pl.pallas_call(
        body,
        grid=(grid,),
        in_specs=[
            pl.BlockSpec((BE, 16), lambda i: (i, 0)),
            pl.BlockSpec((BE, 3), lambda i: (i, 0)),
            eb(),
            full(16, 64), full(64, 64), full(64, 384),
            full(384, _MSG), full(9, _MSG), full(5, 9), full(4, 9),
        ],
        out_specs=[eb(), eb(), eb()],
        out_shape=[jax.ShapeDtypeStruct((_NE, _C), jnp.float32)] * 3,
    )(ef, ev, src, W1, W2, W3, Q, fexp, sha, shb)


def _sc_scatter(m0, m1, m2, dst, z25):
    mesh = plsc.VectorSubcoreMesh(core_axis_name="c", subcore_axis_name="s")

    @functools.partial(
        pl.kernel,
        mesh=mesh,
        compiler_params=_SC_PARAMS,
        out_type=[jax.ShapeDtypeStruct((_NN, _C), jnp.float32)] * 3,
        scratch_types=[
            pltpu.VMEM((_CH,), jnp.int32),
            pltpu.VMEM((_CH,), jnp.int32),
            pltpu.VMEM((_CH, _CW), jnp.float32),
            pltpu.VMEM((_CH, _CW), jnp.float32),
            pltpu.VMEM((25, _CW), jnp.float32),
            pltpu.VMEM_SHARED((_NN, _CW), jnp.float32),
            pltpu.SemaphoreType.DMA,
            pltpu.SemaphoreType.DMA,
            pltpu.SemaphoreType.DMA,
            pltpu.SemaphoreType.DMA,
            pltpu.SemaphoreType.DMA,
            pltpu.SemaphoreType.DMA,
        ],
    )
    def k(m0_hbm, m1_hbm, m2_hbm, dst_hbm, z_hbm, o0_hbm, o1_hbm, o2_hbm,
          ix0, ix1, b0, b1, zbuf, acc, se0, se1, sl0, sl1, sa0, sa1):
        c = lax.axis_index("c")
        sid = lax.axis_index("s")
        slabs = (m0_hbm, m1_hbm, m2_hbm)
        outs = (o0_hbm, o1_hbm, o2_hbm)
        ixb = (ix0, ix1)
        mb = (b0, b1)
        se = (se0, se1)
        sl = (sl0, sl1)
        sa = (sa0, sa1)

        nj = 78 + jnp.where(sid < 2, 1, 0)

        def chunk_off(j):
            return (sid + j * 16) * _CH

        pltpu.sync_copy(z_hbm, zbuf)

        def emit_phase(cv, p):
            pieces = _PIECES[(cv, p)]

            def start_loads(j, b):
                off = chunk_off(j)
                pltpu.async_copy(dst_hbm.at[pl.ds(off, _CH)], ixb[b], se[b])
                for i, (slab, scol, dcol, wdt) in enumerate(pieces):
                    pltpu.async_copy(
                        slabs[slab].at[pl.ds(off, _CH), pl.ds(scol, wdt)],
                        mb[b].at[:, pl.ds(dcol, wdt)],
                        sl[b] if i == 0 else se[b])

            def wait_loads(b):
                pltpu.make_async_copy(
                    dst_hbm.at[pl.ds(0, _CH)], ixb[b], se[b]).wait()
                for i, (slab, scol, dcol, wdt) in enumerate(pieces):
                    pltpu.make_async_copy(
                        slabs[slab].at[pl.ds(0, _CH), pl.ds(scol, wdt)],
                        mb[b].at[:, pl.ds(dcol, wdt)],
                        sl[b] if i == 0 else se[b]).wait()

            def zbody(kk, carry):
                pltpu.sync_copy(zbuf, acc.at[pl.ds(sid * 625 + kk * 25, 25)])
                return carry

            lax.fori_loop(0, 25, zbody, 0)
            plsc.subcore_barrier()

            start_loads(0, 0)

            def iteration(j, b):
                nb = 1 - b
                wait_loads(b)
                pltpu.async_copy(mb[b], acc.at[ixb[b]], sa[b], add=True)

                @pl.when(j + 1 < nj)
                def _():
                    @pl.when(j >= 1)
                    def _():
                        pltpu.make_async_copy(mb[nb], acc.at[ixb[nb]],
                                              sa[nb]).wait()
                    start_loads(j + 1, nb)

            def body(j, carry):
                @pl.when(j % 2 == 0)
                def _():
                    iteration(j, 0)

                @pl.when(j % 2 == 1)
                def _():
                    iteration(j, 1)
                return carry

            lax.fori_loop(0, nj, body, 0)
            pltpu.make_async_copy(b0, acc.at[ix0], sa0).wait()
            pltpu.make_async_copy(b1, acc.at[ix1], sa1).wait()
            plsc.subcore_barrier()

            def wbody(kk, carry):
                r0 = sid * 625 + kk * 125
                pltpu.sync_copy(acc.at[pl.ds(r0, 125)], b0.at[pl.ds(0, 125)])
                for slab, scol, dcol, wdt in pieces:
                    pltpu.sync_copy(
                        b0.at[pl.ds(0, 125), pl.ds(dcol, wdt)],
                        outs[slab].at[pl.ds(r0, 125), pl.ds(scol, wdt)])
                return carry

            lax.fori_loop(0, 5, wbody, 0)
            plsc.subcore_barrier()

        for p in range(2):
            @pl.when(c == 0)
            def _():
                emit_phase(0, p)

            @pl.when(c == 1)
            def _():
                emit_phase(1, p)

    return k(m0, m1, m2, dst, z25)


def kernel(node_features, edge_features, edge_vectors, edge_index, W1, W2, W3, P):
    src_idx = edge_index[:, 0]
    dst_idx = edge_index[:, 1]
    src = _sc_gather(node_features, src_idx)
    bf = jnp.bfloat16
    Q = jnp.concatenate(
        [P[0].T @ _EXP0, P[1].T @ _EXP1, P[2].T @ _EXP2], axis=0).astype(bf)
    sha = jnp.concatenate(
        [jnp.asarray(_SH_A), jnp.asarray(_SH_AR), jnp.asarray(_SH_CR)], axis=0)
    shb = jnp.concatenate([jnp.asarray(_SH_B), jnp.asarray(_SH_BR)], axis=0)
    m0, m1, m2 = _tc_messages(
        edge_features, edge_vectors, src, W1.astype(bf), W2.astype(bf),
        W3.astype(bf), Q, jnp.asarray(_FEXP), sha, shb)
    z25 = jnp.zeros((25, _CW), jnp.float32)
    o0, o1, o2 = _sc_scatter(m0, m1, m2, dst_idx, z25)
    return jnp.concatenate([o0, o1, o2[:, 0:32]], axis=1)


# speedup vs baseline: 1.6767x; 1.0867x over previous
import functools

import numpy as np
import jax
import jax.numpy as jnp
from jax import lax
from jax.experimental import pallas as pl
from jax.experimental.pallas import tpu as pltpu
from jax.experimental.pallas import tpu_sc as plsc

_NN = 10000
_NE = 160000
_C = 128
_MSG = 288
_CW = 64
_CH = 128
_NCH = _NE // _CH

_UNITS = {
    (0, 0): (0, 0),
    (1, 0): (0, 64),
    (0, 1): (1, 0),
    (1, 1): (1, 64),
    (0, 2): (2, 0),
}


def _build_expand():
    exp = [np.zeros((32, _MSG), np.float32) for _ in range(3)]
    fexp = np.zeros((9, _MSG), np.float32)
    for j in range(_MSG):
        if j < 32:
            l, m, k = 0, j, 0
        elif j < 128:
            l, m, k = 1, (j - 32) // 3, (j - 32) % 3
        else:
            l, m, k = 2, (j - 128) // 5, (j - 128) % 5
        exp[l][m, j] = 1.0
        fexp[(0, 1, 4)[l] + k, j] = 1.0
    return exp[0], exp[1], exp[2], fexp


_EXP0, _EXP1, _EXP2, _FEXP = _build_expand()


def _build_sh_affine():
    c1 = np.sqrt(3.0)
    c2 = np.sqrt(15.0)
    c2b = np.sqrt(5.0) / 2.0
    A = np.zeros((3, 9), np.float32)
    B = np.zeros((3, 9), np.float32)
    ar = np.zeros((9,), np.float32)
    br = np.zeros((9,), np.float32)
    cr = np.zeros((9,), np.float32)
    ar[0] = 1.0
    br[0] = 1.0
    for j, ax in ((1, 0), (2, 1), (3, 2)):
        A[ax, j] = 1.0
        br[j] = c1
    A[0, 4] = 1.0; B[1, 4] = c2
    A[1, 5] = 1.0; B[2, 5] = c2
    A[2, 6] = 1.0; B[2, 6] = 3.0 * c2b
    cr[6] = -c2b
    A[0, 7] = 1.0; B[2, 7] = c2
    A[0, 8] = 1.0; A[1, 8] = -1.0
    B[0, 8] = c2 / 2.0; B[1, 8] = c2 / 2.0
    return A, B, ar.reshape(1, 9), br.reshape(1, 9), cr.reshape(1, 9)


_SH_A, _SH_B, _SH_AR, _SH_BR, _SH_CR = _build_sh_affine()

_SC_PARAMS = pltpu.CompilerParams(use_tc_tiling_on_sc=False,
                                  needs_layout_passes=False)


def _sc_gather(table, idx):
    mesh = plsc.VectorSubcoreMesh(core_axis_name="c", subcore_axis_name="s")

    @functools.partial(
        pl.kernel,
        mesh=mesh,
        compiler_params=_SC_PARAMS,
        out_type=jax.ShapeDtypeStruct((_NE, _C), jnp.float32),
        scratch_types=[
            pltpu.VMEM((_CH,), jnp.int32),
            pltpu.VMEM((_CH,), jnp.int32),
            pltpu.VMEM((_CH, _C), jnp.float32),
            pltpu.VMEM((_CH, _C), jnp.float32),
            pltpu.SemaphoreType.DMA,
            pltpu.SemaphoreType.DMA,
            pltpu.SemaphoreType.DMA,
            pltpu.SemaphoreType.DMA,
            pltpu.SemaphoreType.DMA,
            pltpu.SemaphoreType.DMA,
        ],
    )
    def k(table_hbm, idx_hbm, out_hbm, ix0, ix1, rw0, rw1,
          se0, se1, sg0, sg1, ss0, ss1):
        wid = lax.axis_index("s") * 2 + lax.axis_index("c")
        ixb = (ix0, ix1)
        rwb = (rw0, rw1)
        se = (se0, se1)
        sg = (sg0, sg1)
        ss = (ss0, ss1)
        nj = 39 + jnp.where(wid < 2, 1, 0)

        def chunk_off(j):
            return (wid + j * 32) * _CH

        pltpu.async_copy(idx_hbm.at[pl.ds(chunk_off(0), _CH)], ix0, se0)

        def iteration(j, b):
            nb = 1 - b
            pltpu.make_async_copy(
                idx_hbm.at[pl.ds(0, _CH)], ixb[b], se[b]).wait()

            @pl.when(j >= 2)
            def _():
                pltpu.make_async_copy(
                    rwb[b], out_hbm.at[pl.ds(0, _CH)], ss[b]).wait()

            pltpu.async_copy(table_hbm.at[ixb[b]], rwb[b], sg[b])

            @pl.when(j + 1 < nj)
            def _():
                pltpu.async_copy(
                    idx_hbm.at[pl.ds(chunk_off(j + 1), _CH)], ixb[nb], se[nb])

            pltpu.make_async_copy(table_hbm.at[ixb[b]], rwb[b], sg[b]).wait()
            pltpu.async_copy(rwb[b], out_hbm.at[pl.ds(chunk_off(j), _CH)],
                             ss[b])

        def body(j, carry):
            @pl.when(j % 2 == 0)
            def _():
                iteration(j, 0)

            @pl.when(j % 2 == 1)
            def _():
                iteration(j, 1)
            return carry

        lax.fori_loop(0, nj, body, 0)
        pltpu.make_async_copy(rw0, out_hbm.at[pl.ds(0, _CH)], ss0).wait()
        pltpu.make_async_copy(rw1, out_hbm.at[pl.ds(0, _CH)], ss1).wait()

    return k(table, idx)


def _tc_messages(ef, ev, src, W1, W2, W3, Q, fexp, sha, shb):
    BE = 2000
    grid = _NE // BE

    def body(ef_r, ev_r, src_r, W1_r, W2_r, W3_r, Q_r, f_r, a_r, b_r,
             o0_r, o1_r, o2_r):
        bf = jnp.bfloat16
        f32 = jnp.float32
        h = jax.nn.silu(jnp.dot(ef_r[...].astype(bf), W1_r[...],
                                preferred_element_type=f32))
        h = jax.nn.silu(jnp.dot(h.astype(bf), W2_r[...],
                                preferred_element_type=f32))
        w = jax.nn.silu(jnp.dot(h.astype(bf), W3_r[...],
                                preferred_element_type=f32))
        s = src_r[...]
        g = w * jnp.concatenate([s, s, s], axis=1)
        acc = jnp.dot(g.astype(bf), Q_r[...],
                      preferred_element_type=f32)
        v = ev_r[...]
        n = jnp.sqrt(jnp.sum(v * v, axis=1, keepdims=True))
        vn = v / jnp.maximum(n, 1e-9)
        u = vn @ a_r[0:3] + a_r[3:4]
        t = vn @ b_r[0:3] + b_r[3:4]
        y9 = u * t + a_r[4:5]
        y288 = y9 @ f_r[...]
        res = acc * y288
        o0_r[...] = res[:, 0:128]
        o1_r[...] = res[:, 128:256]
        o2_r[:, 0:32] = res[:, 256:288]
        o2_r[:, 32:64] = jnp.zeros((BE, 32), jnp.float32)

    full = lambda a, b: pl.BlockSpec((a, b), lambda i: (0, 0))
    eb = lambda: pl.BlockSpec((BE, _C), lambda i: (i, 0))
    return pl.pallas_call(
        body,
        grid=(grid,),
        in_specs=[
            pl.BlockSpec((BE, 16), lambda i: (i, 0)),
            pl.BlockSpec((BE, 3), lambda i: (i, 0)),
            eb(),
            full(16, 64), full(64, 64), full(64, 384),
            full(384, _MSG), full(9, _MSG), full(5, 9), full(4, 9),
        ],
        out_specs=[eb(), eb(), eb()],
        out_shape=[jax.ShapeDtypeStruct((_NE, _C), jnp.float32)] * 3,
    )(ef, ev, src, W1, W2, W3, Q, fexp, sha, shb)


def _sc_scatter(m0, m1, m2, dst, z25):
    mesh = plsc.VectorSubcoreMesh(core_axis_name="c", subcore_axis_name="s")

    @functools.partial(
        pl.kernel,
        mesh=mesh,
        compiler_params=_SC_PARAMS,
        out_type=[jax.ShapeDtypeStruct((_NN, _C), jnp.float32)] * 3,
        scratch_types=[
            pltpu.VMEM((_CH,), jnp.int32),
            pltpu.VMEM((_CH,), jnp.int32),
            pltpu.VMEM((_CH, _CW), jnp.float32),
            pltpu.VMEM((_CH, _CW), jnp.float32),
            pltpu.VMEM((25, _CW), jnp.float32),
            pltpu.VMEM_SHARED((_NN, _CW), jnp.float32),
            pltpu.SemaphoreType.DMA,
            pltpu.SemaphoreType.DMA,
            pltpu.SemaphoreType.DMA,
            pltpu.SemaphoreType.DMA,
            pltpu.SemaphoreType.DMA,
            pltpu.SemaphoreType.DMA,
        ],
    )
    def k(m0_hbm, m1_hbm, m2_hbm, dst_hbm, z_hbm, o0_hbm, o1_hbm, o2_hbm,
          ix0, ix1, b0, b1, zbuf, acc, se0, se1, sl0, sl1, sa0, sa1):
        c = lax.axis_index("c")
        sid = lax.axis_index("s")
        slabs = (m0_hbm, m1_hbm, m2_hbm)
        outs = (o0_hbm, o1_hbm, o2_hbm)
        ixb = (ix0, ix1)
        mb = (b0, b1)
        se = (se0, se1)
        sl = (sl0, sl1)
        sa = (sa0, sa1)

        nj = 78 + jnp.where(sid < 2, 1, 0)

        def chunk_off(j):
            return (sid + j * 16) * _CH

        pltpu.sync_copy(z_hbm, zbuf)

        def emit_phase(cv, p):
            slab, scol = _UNITS[(cv, p)]

            def start_loads(j, b):
                off = chunk_off(j)
                pltpu.async_copy(dst_hbm.at[pl.ds(off, _CH)], ixb[b], se[b])
                pltpu.async_copy(
                    slabs[slab].at[pl.ds(off, _CH), pl.ds(scol, _CW)],
                    mb[b], sl[b])

            def wait_loads(b):
                pltpu.make_async_copy(
                    dst_hbm.at[pl.ds(0, _CH)], ixb[b], se[b]).wait()
                pltpu.make_async_copy(
                    slabs[slab].at[pl.ds(0, _CH), pl.ds(scol, _CW)],
                    mb[b], sl[b]).wait()

            def zbody(kk, carry):
                pltpu.sync_copy(zbuf, acc.at[pl.ds(sid * 625 + kk * 25, 25)])
                return carry

            lax.fori_loop(0, 25, zbody, 0)
            plsc.subcore_barrier()

            start_loads(0, 0)

            def iteration(j, b):
                nb = 1 - b
                wait_loads(b)
                pltpu.async_copy(mb[b], acc.at[ixb[b]], sa[b], add=True)

                @pl.when(j + 1 < nj)
                def _():
                    @pl.when(j >= 1)
                    def _():
                        pltpu.make_async_copy(mb[nb], acc.at[ixb[nb]],
                                              sa[nb]).wait()
                    start_loads(j + 1, nb)

            def body(j, carry):
                @pl.when(j % 2 == 0)
                def _():
                    iteration(j, 0)

                @pl.when(j % 2 == 1)
                def _():
                    iteration(j, 1)
                return carry

            lax.fori_loop(0, nj, body, 0)
            pltpu.make_async_copy(b0, acc.at[ix0], sa0).wait()
            pltpu.make_async_copy(b1, acc.at[ix1], sa1).wait()
            plsc.subcore_barrier()

            def wbody(kk, carry):
                r0 = sid * 625 + kk * 125
                pltpu.sync_copy(acc.at[pl.ds(r0, 125)], b0.at[pl.ds(0, 125)])
                pltpu.sync_copy(
                    b0.at[pl.ds(0, 125)],
                    outs[slab].at[pl.ds(r0, 125), pl.ds(scol, _CW)])
                return carry

            lax.fori_loop(0, 5, wbody, 0)
            plsc.subcore_barrier()

        for p in range(2):
            @pl.when(c == 0)
            def _():
                emit_phase(0, p)

            @pl.when(c == 1)
            def _():
                emit_phase(1, p)

        @pl.when(c == 0)
        def _():
            emit_phase(0, 2)

    return k(m0, m1, m2, dst, z25)


def kernel(node_features, edge_features, edge_vectors, edge_index, W1, W2, W3, P):
    src_idx = edge_index[:, 0]
    dst_idx = edge_index[:, 1]
    src = _sc_gather(node_features, src_idx)
    bf = jnp.bfloat16
    Q = jnp.concatenate(
        [P[0].T @ _EXP0, P[1].T @ _EXP1, P[2].T @ _EXP2], axis=0).astype(bf)
    sha = jnp.concatenate(
        [jnp.asarray(_SH_A), jnp.asarray(_SH_AR), jnp.asarray(_SH_CR)], axis=0)
    shb = jnp.concatenate([jnp.asarray(_SH_B), jnp.asarray(_SH_BR)], axis=0)
    m0, m1, m2 = _tc_messages(
        edge_features, edge_vectors, src, W1.astype(bf), W2.astype(bf),
        W3.astype(bf), Q, jnp.asarray(_FEXP), sha, shb)
    z25 = jnp.zeros((25, _CW), jnp.float32)
    o0, o1, o2 = _sc_scatter(m0, m1, m2, dst_idx, z25)
    return jnp.concatenate([o0, o1, o2[:, 0:32]], axis=1)


# speedup vs baseline: 1.7947x; 1.0704x over previous
import functools

import numpy as np
import jax
import jax.numpy as jnp
from jax import lax
from jax.experimental import pallas as pl
from jax.experimental.pallas import tpu as pltpu
from jax.experimental.pallas import tpu_sc as plsc

_NN = 10000
_NE = 160000
_C = 128
_MSG = 288
_CW = 64
_CH = 128
_NCH = _NE // _CH

_UNITS = {
    (0, 0): (0, 0),
    (1, 0): (0, 64),
    (0, 1): (1, 0),
    (1, 1): (1, 64),
    (0, 2): (2, 0),
}


def _build_expand():
    exp = [np.zeros((32, _MSG), np.float32) for _ in range(3)]
    fexp = np.zeros((9, _MSG), np.float32)
    for j in range(_MSG):
        if j < 32:
            l, m, k = 0, j, 0
        elif j < 128:
            l, m, k = 1, (j - 32) // 3, (j - 32) % 3
        else:
            l, m, k = 2, (j - 128) // 5, (j - 128) % 5
        exp[l][m, j] = 1.0
        fexp[(0, 1, 4)[l] + k, j] = 1.0
    return exp[0], exp[1], exp[2], fexp


_EXP0, _EXP1, _EXP2, _FEXP = _build_expand()


def _build_sh_affine():
    c1 = np.sqrt(3.0)
    c2 = np.sqrt(15.0)
    c2b = np.sqrt(5.0) / 2.0
    A = np.zeros((3, 9), np.float32)
    B = np.zeros((3, 9), np.float32)
    ar = np.zeros((9,), np.float32)
    br = np.zeros((9,), np.float32)
    cr = np.zeros((9,), np.float32)
    ar[0] = 1.0
    br[0] = 1.0
    for j, ax in ((1, 0), (2, 1), (3, 2)):
        A[ax, j] = 1.0
        br[j] = c1
    A[0, 4] = 1.0; B[1, 4] = c2
    A[1, 5] = 1.0; B[2, 5] = c2
    A[2, 6] = 1.0; B[2, 6] = 3.0 * c2b
    cr[6] = -c2b
    A[0, 7] = 1.0; B[2, 7] = c2
    A[0, 8] = 1.0; A[1, 8] = -1.0
    B[0, 8] = c2 / 2.0; B[1, 8] = c2 / 2.0
    return A, B, ar.reshape(1, 9), br.reshape(1, 9), cr.reshape(1, 9)


_SH_A, _SH_B, _SH_AR, _SH_BR, _SH_CR = _build_sh_affine()

_SC_PARAMS = pltpu.CompilerParams(use_tc_tiling_on_sc=False,
                                  needs_layout_passes=False)


def _sc_gather(table, idx):
    mesh = plsc.VectorSubcoreMesh(core_axis_name="c", subcore_axis_name="s")

    @functools.partial(
        pl.kernel,
        mesh=mesh,
        compiler_params=_SC_PARAMS,
        out_type=jax.ShapeDtypeStruct((_NE, _C), jnp.float32),
        scratch_types=[
            pltpu.VMEM((_CH,), jnp.int32),
            pltpu.VMEM((_CH,), jnp.int32),
            pltpu.VMEM((_CH, _C), jnp.float32),
            pltpu.VMEM((_CH, _C), jnp.float32),
            pltpu.SemaphoreType.DMA,
            pltpu.SemaphoreType.DMA,
            pltpu.SemaphoreType.DMA,
            pltpu.SemaphoreType.DMA,
            pltpu.SemaphoreType.DMA,
            pltpu.SemaphoreType.DMA,
        ],
    )
    def k(table_hbm, idx_hbm, out_hbm, ix0, ix1, rw0, rw1,
          se0, se1, sg0, sg1, ss0, ss1):
        wid = lax.axis_index("s") * 2 + lax.axis_index("c")
        ixb = (ix0, ix1)
        rwb = (rw0, rw1)
        se = (se0, se1)
        sg = (sg0, sg1)
        ss = (ss0, ss1)
        nj = 39 + jnp.where(wid < 2, 1, 0)

        def chunk_off(j):
            return (wid + j * 32) * _CH

        pltpu.async_copy(idx_hbm.at[pl.ds(chunk_off(0), _CH)], ix0, se0)

        def iteration(j, b):
            nb = 1 - b
            pltpu.make_async_copy(
                idx_hbm.at[pl.ds(0, _CH)], ixb[b], se[b]).wait()

            @pl.when(j >= 2)
            def _():
                pltpu.make_async_copy(
                    rwb[b], out_hbm.at[pl.ds(0, _CH)], ss[b]).wait()

            pltpu.async_copy(table_hbm.at[ixb[b]], rwb[b], sg[b])

            @pl.when(j + 1 < nj)
            def _():
                pltpu.async_copy(
                    idx_hbm.at[pl.ds(chunk_off(j + 1), _CH)], ixb[nb], se[nb])

            pltpu.make_async_copy(table_hbm.at[ixb[b]], rwb[b], sg[b]).wait()
            pltpu.async_copy(rwb[b], out_hbm.at[pl.ds(chunk_off(j), _CH)],
                             ss[b])

        def body(j, carry):
            @pl.when(j % 2 == 0)
            def _():
                iteration(j, 0)

            @pl.when(j % 2 == 1)
            def _():
                iteration(j, 1)
            return carry

        lax.fori_loop(0, nj, body, 0)
        pltpu.make_async_copy(rw0, out_hbm.at[pl.ds(0, _CH)], ss0).wait()
        pltpu.make_async_copy(rw1, out_hbm.at[pl.ds(0, _CH)], ss1).wait()

    return k(table, idx)


def _tc_messages(ef, ev, src, W1, W2, W3, Q, fexp, sha, shb):
    BE = 2000
    grid = _NE // BE

    def body(ef_r, ev_r, src_r, W1_r, W2_r, W3_r, Q_r, f_r, a_r, b_r,
             o0_r, o1_r, o2_r):
        bf = jnp.bfloat16
        f32 = jnp.float32
        h = jax.nn.silu(jnp.dot(ef_r[...].astype(bf), W1_r[...],
                                preferred_element_type=f32))
        h = jax.nn.silu(jnp.dot(h.astype(bf), W2_r[...],
                                preferred_element_type=f32))
        w = jax.nn.silu(jnp.dot(h.astype(bf), W3_r[...],
                                preferred_element_type=f32))
        s = src_r[...]
        g = w * jnp.concatenate([s, s, s], axis=1)
        acc = jnp.dot(g.astype(bf), Q_r[...],
                      preferred_element_type=f32)
        v = ev_r[...]
        n = jnp.sqrt(jnp.sum(v * v, axis=1, keepdims=True))
        vn = v / jnp.maximum(n, 1e-9)
        u = vn @ a_r[0:3] + a_r[3:4]
        t = vn @ b_r[0:3] + b_r[3:4]
        y9 = u * t + a_r[4:5]
        y288 = y9 @ f_r[...]
        res = acc * y288
        o0_r[...] = res[:, 0:128]
        o1_r[...] = res[:, 128:256]
        o2_r[:, 0:32] = res[:, 256:288]
        o2_r[:, 32:64] = jnp.zeros((BE, 32), jnp.float32)

    full = lambda a, b: pl.BlockSpec((a, b), lambda i: (0, 0))
    eb = lambda: pl.BlockSpec((BE, _C), lambda i: (i, 0))
    return pl.pallas_call(
        body,
        grid=(grid,),
        in_specs=[
            pl.BlockSpec((BE, 16), lambda i: (i, 0)),
            pl.BlockSpec((BE, 3), lambda i: (i, 0)),
            eb(),
            full(16, 64), full(64, 64), full(64, 384),
            full(384, _MSG), full(9, _MSG), full(5, 9), full(4, 9),
        ],
        out_specs=[eb(), eb(), eb()],
        out_shape=[jax.ShapeDtypeStruct((_NE, _C), jnp.float32)] * 3,
    )(ef, ev, src, W1, W2, W3, Q, fexp, sha, shb)


def _tc_assemble(o0, o1, o2):
    BR = 2000

    def body(a_r, b_r, c_r, out_r):
        out_r[:, 0:128] = a_r[...]
        out_r[:, 128:256] = b_r[...]
        out_r[:, 256:288] = c_r[:, 0:32]

    spec = pl.BlockSpec((BR, _C), lambda i: (i, 0))
    return pl.pallas_call(
        body,
        grid=(_NN // BR,),
        in_specs=[spec, spec, spec],
        out_specs=pl.BlockSpec((BR, _MSG), lambda i: (i, 0)),
        out_shape=jax.ShapeDtypeStruct((_NN, _MSG), jnp.float32),
    )(o0, o1, o2)


def _sc_scatter(m0, m1, m2, dst, z25):
    mesh = plsc.VectorSubcoreMesh(core_axis_name="c", subcore_axis_name="s")

    @functools.partial(
        pl.kernel,
        mesh=mesh,
        compiler_params=_SC_PARAMS,
        out_type=[jax.ShapeDtypeStruct((_NN, _C), jnp.float32)] * 3,
        scratch_types=[
            pltpu.VMEM((_CH,), jnp.int32),
            pltpu.VMEM((_CH,), jnp.int32),
            pltpu.VMEM((_CH, _CW), jnp.float32),
            pltpu.VMEM((_CH, _CW), jnp.float32),
            pltpu.VMEM((25, _CW), jnp.float32),
            pltpu.VMEM_SHARED((_NN, _CW), jnp.float32),
            pltpu.SemaphoreType.DMA,
            pltpu.SemaphoreType.DMA,
            pltpu.SemaphoreType.DMA,
            pltpu.SemaphoreType.DMA,
            pltpu.SemaphoreType.DMA,
            pltpu.SemaphoreType.DMA,
        ],
    )
    def k(m0_hbm, m1_hbm, m2_hbm, dst_hbm, z_hbm, o0_hbm, o1_hbm, o2_hbm,
          ix0, ix1, b0, b1, zbuf, acc, se0, se1, sl0, sl1, sa0, sa1):
        c = lax.axis_index("c")
        sid = lax.axis_index("s")
        slabs = (m0_hbm, m1_hbm, m2_hbm)
        outs = (o0_hbm, o1_hbm, o2_hbm)
        ixb = (ix0, ix1)
        mb = (b0, b1)
        se = (se0, se1)
        sl = (sl0, sl1)
        sa = (sa0, sa1)

        nj = 78 + jnp.where(sid < 2, 1, 0)

        def chunk_off(j):
            return (sid + j * 16) * _CH

        pltpu.sync_copy(z_hbm, zbuf)

        def emit_phase(cv, p):
            slab, scol = _UNITS[(cv, p)]

            def start_loads(j, b):
                off = chunk_off(j)
                pltpu.async_copy(dst_hbm.at[pl.ds(off, _CH)], ixb[b], se[b])
                pltpu.async_copy(
                    slabs[slab].at[pl.ds(off, _CH), pl.ds(scol, _CW)],
                    mb[b], sl[b])

            def wait_loads(b):
                pltpu.make_async_copy(
                    dst_hbm.at[pl.ds(0, _CH)], ixb[b], se[b]).wait()
                pltpu.make_async_copy(
                    slabs[slab].at[pl.ds(0, _CH), pl.ds(scol, _CW)],
                    mb[b], sl[b]).wait()

            def zbody(kk, carry):
                pltpu.sync_copy(zbuf, acc.at[pl.ds(sid * 625 + kk * 25, 25)])
                return carry

            lax.fori_loop(0, 25, zbody, 0)
            plsc.subcore_barrier()

            start_loads(0, 0)

            def iteration(j, b):
                nb = 1 - b
                wait_loads(b)
                pltpu.async_copy(mb[b], acc.at[ixb[b]], sa[b], add=True)

                @pl.when(j + 1 < nj)
                def _():
                    @pl.when(j >= 1)
                    def _():
                        pltpu.make_async_copy(mb[nb], acc.at[ixb[nb]],
                                              sa[nb]).wait()
                    start_loads(j + 1, nb)

            def body(j, carry):
                @pl.when(j % 2 == 0)
                def _():
                    iteration(j, 0)

                @pl.when(j % 2 == 1)
                def _():
                    iteration(j, 1)
                return carry

            lax.fori_loop(0, nj, body, 0)
            pltpu.make_async_copy(b0, acc.at[ix0], sa0).wait()
            pltpu.make_async_copy(b1, acc.at[ix1], sa1).wait()
            plsc.subcore_barrier()

            def wbody(kk, carry):
                r0 = sid * 625 + kk * 125
                pltpu.sync_copy(acc.at[pl.ds(r0, 125)], b0.at[pl.ds(0, 125)])
                pltpu.sync_copy(
                    b0.at[pl.ds(0, 125)],
                    outs[slab].at[pl.ds(r0, 125), pl.ds(scol, _CW)])
                return carry

            lax.fori_loop(0, 5, wbody, 0)
            plsc.subcore_barrier()

        for p in range(2):
            @pl.when(c == 0)
            def _():
                emit_phase(0, p)

            @pl.when(c == 1)
            def _():
                emit_phase(1, p)

        @pl.when(c == 0)
        def _():
            emit_phase(0, 2)

    return k(m0, m1, m2, dst, z25)


def kernel(node_features, edge_features, edge_vectors, edge_index, W1, W2, W3, P):
    src_idx = edge_index[:, 0]
    dst_idx = edge_index[:, 1]
    src = _sc_gather(node_features, src_idx)
    bf = jnp.bfloat16
    Q = jnp.concatenate(
        [P[0].T @ _EXP0, P[1].T @ _EXP1, P[2].T @ _EXP2], axis=0).astype(bf)
    sha = jnp.concatenate(
        [jnp.asarray(_SH_A), jnp.asarray(_SH_AR), jnp.asarray(_SH_CR)], axis=0)
    shb = jnp.concatenate([jnp.asarray(_SH_B), jnp.asarray(_SH_BR)], axis=0)
    m0, m1, m2 = _tc_messages(
        edge_features, edge_vectors, src, W1.astype(bf), W2.astype(bf),
        W3.astype(bf), Q, jnp.asarray(_FEXP), sha, shb)
    z25 = jnp.zeros((25, _CW), jnp.float32)
    o0, o1, o2 = _sc_scatter(m0, m1, m2, dst_idx, z25)
    return _tc_assemble(o0, o1, o2)


# speedup vs baseline: 1.8377x; 1.0240x over previous
import functools

import numpy as np
import jax
import jax.numpy as jnp
from jax import lax
from jax.experimental import pallas as pl
from jax.experimental.pallas import tpu as pltpu
from jax.experimental.pallas import tpu_sc as plsc

_NN = 10000
_NE = 160000
_C = 128
_MSG = 288
_CW = 64
_CH = 128
_NCH = _NE // _CH

_UNITS = {
    (0, 0): (0, 0, 64),
    (1, 0): (0, 64, 64),
    (0, 1): (1, 0, 64),
    (1, 1): (1, 64, 64),
    (0, 2): (2, 0, 16),
    (1, 2): (2, 16, 16),
}


def _build_expand():
    exp = [np.zeros((32, _MSG), np.float32) for _ in range(3)]
    fexp = np.zeros((9, _MSG), np.float32)
    for j in range(_MSG):
        if j < 32:
            l, m, k = 0, j, 0
        elif j < 128:
            l, m, k = 1, (j - 32) // 3, (j - 32) % 3
        else:
            l, m, k = 2, (j - 128) // 5, (j - 128) % 5
        exp[l][m, j] = 1.0
        fexp[(0, 1, 4)[l] + k, j] = 1.0
    return exp[0], exp[1], exp[2], fexp


_EXP0, _EXP1, _EXP2, _FEXP = _build_expand()


def _build_sh_affine():
    c1 = np.sqrt(3.0)
    c2 = np.sqrt(15.0)
    c2b = np.sqrt(5.0) / 2.0
    A = np.zeros((3, 9), np.float32)
    B = np.zeros((3, 9), np.float32)
    ar = np.zeros((9,), np.float32)
    br = np.zeros((9,), np.float32)
    cr = np.zeros((9,), np.float32)
    ar[0] = 1.0
    br[0] = 1.0
    for j, ax in ((1, 0), (2, 1), (3, 2)):
        A[ax, j] = 1.0
        br[j] = c1
    A[0, 4] = 1.0; B[1, 4] = c2
    A[1, 5] = 1.0; B[2, 5] = c2
    A[2, 6] = 1.0; B[2, 6] = 3.0 * c2b
    cr[6] = -c2b
    A[0, 7] = 1.0; B[2, 7] = c2
    A[0, 8] = 1.0; A[1, 8] = -1.0
    B[0, 8] = c2 / 2.0; B[1, 8] = c2 / 2.0
    return A, B, ar.reshape(1, 9), br.reshape(1, 9), cr.reshape(1, 9)


_SH_A, _SH_B, _SH_AR, _SH_BR, _SH_CR = _build_sh_affine()

_SC_PARAMS = pltpu.CompilerParams(use_tc_tiling_on_sc=False,
                                  needs_layout_passes=False)


def _sc_gather(table, idx):
    mesh = plsc.VectorSubcoreMesh(core_axis_name="c", subcore_axis_name="s")

    @functools.partial(
        pl.kernel,
        mesh=mesh,
        compiler_params=_SC_PARAMS,
        out_type=jax.ShapeDtypeStruct((_NE, _C), jnp.float32),
        scratch_types=[
            pltpu.VMEM((_CH,), jnp.int32),
            pltpu.VMEM((_CH,), jnp.int32),
            pltpu.VMEM((_CH, _C), jnp.float32),
            pltpu.VMEM((_CH, _C), jnp.float32),
            pltpu.SemaphoreType.DMA,
            pltpu.SemaphoreType.DMA,
            pltpu.SemaphoreType.DMA,
            pltpu.SemaphoreType.DMA,
            pltpu.SemaphoreType.DMA,
            pltpu.SemaphoreType.DMA,
        ],
    )
    def k(table_hbm, idx_hbm, out_hbm, ix0, ix1, rw0, rw1,
          se0, se1, sg0, sg1, ss0, ss1):
        wid = lax.axis_index("s") * 2 + lax.axis_index("c")
        ixb = (ix0, ix1)
        rwb = (rw0, rw1)
        se = (se0, se1)
        sg = (sg0, sg1)
        ss = (ss0, ss1)
        nj = 39 + jnp.where(wid < 2, 1, 0)

        def chunk_off(j):
            return (wid + j * 32) * _CH

        pltpu.async_copy(idx_hbm.at[pl.ds(chunk_off(0), _CH)], ix0, se0)

        def iteration(j, b):
            nb = 1 - b
            pltpu.make_async_copy(
                idx_hbm.at[pl.ds(0, _CH)], ixb[b], se[b]).wait()

            @pl.when(j >= 2)
            def _():
                pltpu.make_async_copy(
                    rwb[b], out_hbm.at[pl.ds(0, _CH)], ss[b]).wait()

            pltpu.async_copy(table_hbm.at[ixb[b]], rwb[b], sg[b])

            @pl.when(j + 1 < nj)
            def _():
                pltpu.async_copy(
                    idx_hbm.at[pl.ds(chunk_off(j + 1), _CH)], ixb[nb], se[nb])

            pltpu.make_async_copy(table_hbm.at[ixb[b]], rwb[b], sg[b]).wait()
            pltpu.async_copy(rwb[b], out_hbm.at[pl.ds(chunk_off(j), _CH)],
                             ss[b])

        def body(j, carry):
            @pl.when(j % 2 == 0)
            def _():
                iteration(j, 0)

            @pl.when(j % 2 == 1)
            def _():
                iteration(j, 1)
            return carry

        lax.fori_loop(0, nj, body, 0)
        pltpu.make_async_copy(rw0, out_hbm.at[pl.ds(0, _CH)], ss0).wait()
        pltpu.make_async_copy(rw1, out_hbm.at[pl.ds(0, _CH)], ss1).wait()

    return k(table, idx)


def _tc_messages(ef, ev, src, W1, W2, W3, Q, fexp, sha, shb):
    BE = 2000
    grid = _NE // BE

    def body(ef_r, ev_r, src_r, W1_r, W2_r, W3_r, Q_r, f_r, a_r, b_r,
             o0_r, o1_r, o2_r):
        bf = jnp.bfloat16
        f32 = jnp.float32
        h = jax.nn.silu(jnp.dot(ef_r[...].astype(bf), W1_r[...],
                                preferred_element_type=f32))
        h = jax.nn.silu(jnp.dot(h.astype(bf), W2_r[...],
                                preferred_element_type=f32))
        w = jax.nn.silu(jnp.dot(h.astype(bf), W3_r[...],
                                preferred_element_type=f32))
        s = src_r[...]
        g = w * jnp.concatenate([s, s, s], axis=1)
        acc = jnp.dot(g.astype(bf), Q_r[...],
                      preferred_element_type=f32)
        v = ev_r[...]
        n = jnp.sqrt(jnp.sum(v * v, axis=1, keepdims=True))
        vn = v / jnp.maximum(n, 1e-9)
        u = vn @ a_r[0:3] + a_r[3:4]
        t = vn @ b_r[0:3] + b_r[3:4]
        y9 = u * t + a_r[4:5]
        y288 = y9 @ f_r[...]
        res = acc * y288
        o0_r[...] = res[:, 0:128]
        o1_r[...] = res[:, 128:256]
        o2_r[:, 0:32] = res[:, 256:288]

    full = lambda a, b: pl.BlockSpec((a, b), lambda i: (0, 0))
    eb = lambda: pl.BlockSpec((BE, _C), lambda i: (i, 0))
    return pl.pallas_call(
        body,
        grid=(grid,),
        in_specs=[
            pl.BlockSpec((BE, 16), lambda i: (i, 0)),
            pl.BlockSpec((BE, 3), lambda i: (i, 0)),
            eb(),
            full(16, 64), full(64, 64), full(64, 384),
            full(384, _MSG), full(9, _MSG), full(5, 9), full(4, 9),
        ],
        out_specs=[eb(), eb(), eb()],
        out_shape=[jax.ShapeDtypeStruct((_NE, _C), jnp.float32)] * 3,
    )(ef, ev, src, W1, W2, W3, Q, fexp, sha, shb)


def _tc_assemble(o0, o1, o2):
    BR = 2000

    def body(a_r, b_r, c_r, out_r):
        out_r[:, 0:128] = a_r[...]
        out_r[:, 128:256] = b_r[...]
        out_r[:, 256:288] = c_r[:, 0:32]

    spec = pl.BlockSpec((BR, _C), lambda i: (i, 0))
    return pl.pallas_call(
        body,
        grid=(_NN // BR,),
        in_specs=[spec, spec, spec],
        out_specs=pl.BlockSpec((BR, _MSG), lambda i: (i, 0)),
        out_shape=jax.ShapeDtypeStruct((_NN, _MSG), jnp.float32),
    )(o0, o1, o2)


def _sc_scatter(m0, m1, m2, dst, z25):
    mesh = plsc.VectorSubcoreMesh(core_axis_name="c", subcore_axis_name="s")

    @functools.partial(
        pl.kernel,
        mesh=mesh,
        compiler_params=_SC_PARAMS,
        out_type=[jax.ShapeDtypeStruct((_NN, _C), jnp.float32)] * 3,
        scratch_types=[
            pltpu.VMEM((_CH,), jnp.int32),
            pltpu.VMEM((_CH,), jnp.int32),
            pltpu.VMEM((_CH, _CW), jnp.float32),
            pltpu.VMEM((_CH, _CW), jnp.float32),
            pltpu.VMEM((_CH, 16), jnp.float32),
            pltpu.VMEM((_CH, 16), jnp.float32),
            pltpu.VMEM((25, _CW), jnp.float32),
            pltpu.VMEM_SHARED((_NN, _CW), jnp.float32),
            pltpu.VMEM_SHARED((_NN, 16), jnp.float32),
            pltpu.SemaphoreType.DMA,
            pltpu.SemaphoreType.DMA,
            pltpu.SemaphoreType.DMA,
            pltpu.SemaphoreType.DMA,
            pltpu.SemaphoreType.DMA,
            pltpu.SemaphoreType.DMA,
        ],
    )
    def k(m0_hbm, m1_hbm, m2_hbm, dst_hbm, z_hbm, o0_hbm, o1_hbm, o2_hbm,
          ix0, ix1, b0, b1, n0, n1, zbuf, acc64, acc16,
          se0, se1, sl0, sl1, sa0, sa1):
        c = lax.axis_index("c")
        sid = lax.axis_index("s")
        slabs = (m0_hbm, m1_hbm, m2_hbm)
        outs = (o0_hbm, o1_hbm, o2_hbm)
        ixb = (ix0, ix1)
        se = (se0, se1)
        sl = (sl0, sl1)
        sa = (sa0, sa1)

        nj = 78 + jnp.where(sid < 2, 1, 0)

        def chunk_off(j):
            return (sid + j * 16) * _CH

        pltpu.sync_copy(z_hbm, zbuf)

        def emit_phase(cv, p):
            slab, scol, wdt = _UNITS[(cv, p)]
            mb = (b0, b1) if wdt == _CW else (n0, n1)
            acc = acc64 if wdt == _CW else acc16

            def start_loads(j, b):
                off = chunk_off(j)
                pltpu.async_copy(dst_hbm.at[pl.ds(off, _CH)], ixb[b], se[b])
                pltpu.async_copy(
                    slabs[slab].at[pl.ds(off, _CH), pl.ds(scol, wdt)],
                    mb[b], sl[b])

            def wait_loads(b):
                pltpu.make_async_copy(
                    dst_hbm.at[pl.ds(0, _CH)], ixb[b], se[b]).wait()
                pltpu.make_async_copy(
                    slabs[slab].at[pl.ds(0, _CH), pl.ds(scol, wdt)],
                    mb[b], sl[b]).wait()

            def zbody(kk, carry):
                pltpu.sync_copy(zbuf.at[:, pl.ds(0, wdt)],
                                acc.at[pl.ds(sid * 625 + kk * 25, 25)])
                return carry

            lax.fori_loop(0, 25, zbody, 0)
            plsc.subcore_barrier()

            start_loads(0, 0)

            def iteration(j, b):
                nb = 1 - b
                wait_loads(b)
                pltpu.async_copy(mb[b], acc.at[ixb[b]], sa[b], add=True)

                @pl.when(j + 1 < nj)
                def _():
                    @pl.when(j >= 1)
                    def _():
                        pltpu.make_async_copy(mb[nb], acc.at[ixb[nb]],
                                              sa[nb]).wait()
                    start_loads(j + 1, nb)

            def body(j, carry):
                @pl.when(j % 2 == 0)
                def _():
                    iteration(j, 0)

                @pl.when(j % 2 == 1)
                def _():
                    iteration(j, 1)
                return carry

            lax.fori_loop(0, nj, body, 0)
            pltpu.make_async_copy(mb[0], acc.at[ix0], sa0).wait()
            pltpu.make_async_copy(mb[1], acc.at[ix1], sa1).wait()
            plsc.subcore_barrier()

            def wbody(kk, carry):
                r0 = sid * 625 + kk * 125
                pltpu.sync_copy(acc.at[pl.ds(r0, 125)], mb[0].at[pl.ds(0, 125)])
                pltpu.sync_copy(
                    mb[0].at[pl.ds(0, 125)],
                    outs[slab].at[pl.ds(r0, 125), pl.ds(scol, wdt)])
                return carry

            lax.fori_loop(0, 5, wbody, 0)
            plsc.subcore_barrier()

        for p in range(3):
            @pl.when(c == 0)
            def _():
                emit_phase(0, p)

            @pl.when(c == 1)
            def _():
                emit_phase(1, p)

    return k(m0, m1, m2, dst, z25)


def kernel(node_features, edge_features, edge_vectors, edge_index, W1, W2, W3, P):
    src_idx = edge_index[:, 0]
    dst_idx = edge_index[:, 1]
    src = _sc_gather(node_features, src_idx)
    bf = jnp.bfloat16
    Q = jnp.concatenate(
        [P[0].T @ _EXP0, P[1].T @ _EXP1, P[2].T @ _EXP2], axis=0).astype(bf)
    sha = jnp.concatenate(
        [jnp.asarray(_SH_A), jnp.asarray(_SH_AR), jnp.asarray(_SH_CR)], axis=0)
    shb = jnp.concatenate([jnp.asarray(_SH_B), jnp.asarray(_SH_BR)], axis=0)
    m0, m1, m2 = _tc_messages(
        edge_features, edge_vectors, src, W1.astype(bf), W2.astype(bf),
        W3.astype(bf), Q, jnp.asarray(_FEXP), sha, shb)
    z25 = jnp.zeros((25, _CW), jnp.float32)
    o0, o1, o2 = _sc_scatter(m0, m1, m2, dst_idx, z25)
    return _tc_assemble(o0, o1, o2)
